# baseline (speedup 1.0000x reference)
import jax
import jax.numpy as jnp
from jax import lax
from jax.experimental import pallas as pl
from jax.experimental.pallas import tpu as pltpu
from jax.experimental.pallas import tpu_sc as plsc

NC, NS = 2, 16
NW = NC * NS
CHUNK = 128
NBUF = 4
LEAD = 2


def _make_transpose(vocab, embed):
    nfull = vocab // CHUNK
    tail = vocab - nfull * CHUNK
    nb, nx = divmod(nfull, NW)
    if nb % 2:
        nb -= 1
        nx += NW
    mesh = plsc.VectorSubcoreMesh(
        core_axis_name="c", subcore_axis_name="s",
        num_cores=NC, num_subcores=NS)

    def body(tt_hbm, tp_hbm, tin0, tin1, tout0, tout1, tin_tail,
             is0, is1, os0, os1):
        w = lax.axis_index("s") * NC + lax.axis_index("c")
        lane = lax.iota(jnp.int32, 16)
        lane128 = lane * CHUNK

        vb0 = w * nb

        def in_copy(s, tinb, sem):
            return pltpu.make_async_copy(
                tt_hbm.at[:, pl.ds((vb0 + 2 * s) * CHUNK, 2 * CHUNK)],
                tinb, sem)

        def out_copy(vb, toutb, sem):
            return pltpu.make_async_copy(
                toutb, tp_hbm.at[pl.ds(vb * CHUNK * CHUNK, CHUNK * CHUNK)],
                sem)

        def transpose_block(tinb, toutb, ngroups, loff=0, compact=False):
            if compact:
                shift = ngroups.bit_length() - 1

                @plsc.parallel_loop(0, ngroups * embed, unroll=8)
                def _(t):
                    e = t >> shift
                    lg = t & (ngroups - 1)
                    x = tinb[e, pl.ds(loff + lg * 16, 16)]
                    plsc.store_scatter(toutb, [lane128 + (lg << 11) + e], x)
            else:
                @plsc.parallel_loop(0, ngroups, unroll=2)
                def _(lg):
                    base = lane128 + (lg << 11)
                    col = loff + lg * 16
                    for e in range(embed):
                        x = tinb[e, pl.ds(col, 16)]
                        plsc.store_scatter(toutb, [base + e], x)

        zeros16 = jnp.zeros((16,), jnp.float32)

        @pl.loop(0, CHUNK)
        def _(l):
            for toutb in (tout0, tout1):
                for k in range((CHUNK - embed) // 16):
                    toutb[pl.ds(l * CHUNK + embed + 16 * k, 16)] = zeros16

        nsb = nb // 2
        in_copy(0, tin0, is0).start()
        in_copy(1, tin1, is1).start()

        @pl.loop(0, nsb, step=2)
        def _(s0):
            for b, tinb, isem in ((0, tin0, is0), (1, tin1, is1)):
                s = s0 + b
                in_copy(s, tinb, isem).wait()
                for j, toutb, osem in ((0, tout0, os0), (1, tout1, os1)):
                    @pl.when(s >= 1)
                    def _(s=s, j=j, toutb=toutb, osem=osem):
                        out_copy(vb0 + 2 * (s - 1) + j, toutb, osem).wait()

                    transpose_block(tinb, toutb, 8, loff=j * CHUNK)
                    out_copy(vb0 + 2 * s + j, toutb, osem).start()

                @pl.when(s + 2 < nsb)
                def _(s=s, tinb=tinb, isem=isem):
                    in_copy(s + 2, tinb, isem).start()

        out_copy(vb0 + nb - 2, tout0, os0).wait()
        out_copy(vb0 + nb - 1, tout1, os1).wait()

        @pl.when(w < nx)
        def _():
            vbx = nfull - nx + w
            pltpu.sync_copy(tt_hbm.at[:, pl.ds(vbx * CHUNK, CHUNK)],
                            tin0.at[:, pl.ds(0, CHUNK)])
            transpose_block(tin0, tout0, 8, compact=True)
            pltpu.sync_copy(
                tout0, tp_hbm.at[pl.ds(vbx * CHUNK * CHUNK, CHUNK * CHUNK)])

        if tail:
            @pl.when(w == nx)
            def _():
                pltpu.sync_copy(tt_hbm.at[:, pl.ds(nfull * CHUNK, tail)],
                                tin_tail)
                transpose_block(tin_tail, tout1, tail // 16, compact=True)
                pltpu.sync_copy(
                    tout1.at[pl.ds(0, tail * CHUNK)],
                    tp_hbm.at[pl.ds(nfull * CHUNK * CHUNK, tail * CHUNK)])

    return pl.kernel(
        body,
        out_type=jax.ShapeDtypeStruct((vocab * CHUNK,), jnp.float32),
        mesh=mesh,
        compiler_params=pltpu.CompilerParams(use_tc_tiling_on_sc=True,
                                             needs_layout_passes=False),
        scratch_types=[
            pltpu.VMEM((embed, 2 * CHUNK), jnp.float32),
            pltpu.VMEM((embed, 2 * CHUNK), jnp.float32),
            pltpu.VMEM((CHUNK * CHUNK,), jnp.float32),
            pltpu.VMEM((CHUNK * CHUNK,), jnp.float32),
            pltpu.VMEM((embed, tail if tail else 16), jnp.float32),
            pltpu.SemaphoreType.DMA,
            pltpu.SemaphoreType.DMA,
            pltpu.SemaphoreType.DMA,
            pltpu.SemaphoreType.DMA,
        ],
    )


def _make_gather(vocab, embed, nchunks):
    mesh = plsc.VectorSubcoreMesh(
        core_axis_name="c", subcore_axis_name="s",
        num_cores=NC, num_subcores=NS)

    def body(table_hbm, idx_hbm, out_hbm, idx_v, bufs, gsems, osems):
        wid = lax.axis_index("s") * NC + lax.axis_index("c")
        pltpu.sync_copy(idx_hbm.at[wid], idx_v)
        row_base = wid * nchunks * CHUNK

        def gather_copy(j, slot):
            return pltpu.make_async_copy(
                table_hbm.at[idx_v.at[j]], bufs.at[slot], gsems.at[slot])

        def out_copy(j, slot):
            return pltpu.make_async_copy(
                bufs.at[slot],
                out_hbm.at[pl.ds(row_base + j * CHUNK, CHUNK)],
                osems.at[slot])

        for b in range(LEAD):
            gather_copy(b, b).start()

        @pl.loop(0, nchunks, step=NBUF)
        def _(j0):
            for b in range(NBUF):
                j = j0 + b
                slot_ahead = (b + LEAD) % NBUF

                @pl.when(j >= LEAD)
                def _():
                    out_copy(j - LEAD, slot_ahead).wait()

                @pl.when(j + LEAD < nchunks)
                def _():
                    gather_copy(j + LEAD, slot_ahead).start()

                gather_copy(j, b).wait()
                out_copy(j, b).start()

        for j in range(nchunks - LEAD, nchunks):
            out_copy(j, j % NBUF).wait()

    return pl.kernel(
        body,
        out_type=jax.ShapeDtypeStruct((NW * nchunks * CHUNK, embed),
                                      jnp.float32),
        mesh=mesh,
        compiler_params=pltpu.CompilerParams(use_tc_tiling_on_sc=True),
        scratch_types=[
            pltpu.VMEM((nchunks, CHUNK), jnp.int32),
            pltpu.VMEM((NBUF, CHUNK, embed), jnp.float32),
            pltpu.SemaphoreType.DMA((NBUF,)),
            pltpu.SemaphoreType.DMA((NBUF,)),
        ],
    )


def kernel(inputs, table):
    b, s = inputs.shape
    vocab, embed = table.shape
    total = b * s
    nchunks = total // (NW * CHUNK)
    idx = inputs.reshape(NW, nchunks, CHUNK).astype(jnp.int32)
    table_p = _make_transpose(vocab, embed)(table.T).reshape(vocab, 128)
    out = _make_gather(vocab, 128, nchunks)(table_p, idx)
    return out[:, :embed].reshape(b, s, embed)


# speedup vs baseline: 2.0917x; 2.0917x over previous
import jax
import jax.numpy as jnp
from jax import lax
from jax.experimental import pallas as pl
from jax.experimental.pallas import tpu as pltpu
from jax.experimental.pallas import tpu_sc as plsc

NC, NS = 2, 16
NW = NC * NS
CHUNK = 128
NBUF = 4
LEAD = 2


def _make_transpose(vocab, embed):
    nfull = vocab // CHUNK
    tail = vocab - nfull * CHUNK
    nb, nx = divmod(nfull, NW)
    if nb % 2:
        nb -= 1
        nx += NW
    mesh = plsc.VectorSubcoreMesh(
        core_axis_name="c", subcore_axis_name="s",
        num_cores=NC, num_subcores=NS)

    def body(tt_hbm, tp_hbm, tin0, tin1, tout0, tout1, tin_tail,
             is0, is1, os0, os1):
        w = lax.axis_index("s") * NC + lax.axis_index("c")
        lane = lax.iota(jnp.int32, 16)
        lane128 = lane * CHUNK
        rot = [(lane + d) % 16 for d in range(16)]

        vb0 = w * nb

        def in_copy(s, tinb, sem):
            return pltpu.make_async_copy(
                tt_hbm.at[:, pl.ds((vb0 + 2 * s) * CHUNK, 2 * CHUNK)],
                tinb, sem)

        def out_copy(vb, toutb, sem):
            return pltpu.make_async_copy(
                toutb, tp_hbm.at[pl.ds(vb * CHUNK * CHUNK, CHUNK * CHUNK)],
                sem)

        def transpose_block(tinb, toutb, ngroups, loff=0, compact=False):
            if compact:
                shift = ngroups.bit_length() - 1

                @plsc.parallel_loop(0, ngroups * embed, unroll=8)
                def _(t):
                    e = t >> shift
                    lg = t & (ngroups - 1)
                    x = tinb[e, pl.ds(loff + lg * 16, 16)]
                    plsc.store_scatter(toutb, [lane128 + (lg << 11) + e], x)
            else:
                nsub = embed // 16

                @plsc.parallel_loop(0, ngroups * nsub, unroll=2)
                def _(st):
                    e0 = (st % nsub) * 16
                    l0 = (st // nsub) * 16
                    for d in range(16):
                        x = plsc.load_gather(
                            tinb, [e0 + lane, loff + l0 + rot[d]])
                        plsc.store_scatter(
                            toutb, [(l0 + rot[d]) * CHUNK + e0 + lane], x)

        zeros16 = jnp.zeros((16,), jnp.float32)

        @pl.loop(0, CHUNK)
        def _(l):
            for toutb in (tout0, tout1):
                for k in range((CHUNK - embed) // 16):
                    toutb[pl.ds(l * CHUNK + embed + 16 * k, 16)] = zeros16

        nsb = nb // 2
        in_copy(0, tin0, is0).start()
        in_copy(1, tin1, is1).start()

        @pl.loop(0, nsb, step=2)
        def _(s0):
            for b, tinb, isem in ((0, tin0, is0), (1, tin1, is1)):
                s = s0 + b
                in_copy(s, tinb, isem).wait()
                for j, toutb, osem in ((0, tout0, os0), (1, tout1, os1)):
                    @pl.when(s >= 1)
                    def _(s=s, j=j, toutb=toutb, osem=osem):
                        out_copy(vb0 + 2 * (s - 1) + j, toutb, osem).wait()

                    transpose_block(tinb, toutb, 8, loff=j * CHUNK)
                    out_copy(vb0 + 2 * s + j, toutb, osem).start()

                @pl.when(s + 2 < nsb)
                def _(s=s, tinb=tinb, isem=isem):
                    in_copy(s + 2, tinb, isem).start()

        out_copy(vb0 + nb - 2, tout0, os0).wait()
        out_copy(vb0 + nb - 1, tout1, os1).wait()

        @pl.when(w < nx)
        def _():
            vbx = nfull - nx + w
            pltpu.sync_copy(tt_hbm.at[:, pl.ds(vbx * CHUNK, CHUNK)],
                            tin0.at[:, pl.ds(0, CHUNK)])
            transpose_block(tin0, tout0, 8, compact=True)
            pltpu.sync_copy(
                tout0, tp_hbm.at[pl.ds(vbx * CHUNK * CHUNK, CHUNK * CHUNK)])

        if tail:
            @pl.when(w == nx)
            def _():
                pltpu.sync_copy(tt_hbm.at[:, pl.ds(nfull * CHUNK, tail)],
                                tin_tail)
                transpose_block(tin_tail, tout1, tail // 16, compact=True)
                pltpu.sync_copy(
                    tout1.at[pl.ds(0, tail * CHUNK)],
                    tp_hbm.at[pl.ds(nfull * CHUNK * CHUNK, tail * CHUNK)])

    return pl.kernel(
        body,
        out_type=jax.ShapeDtypeStruct((vocab * CHUNK,), jnp.float32),
        mesh=mesh,
        compiler_params=pltpu.CompilerParams(use_tc_tiling_on_sc=True,
                                             needs_layout_passes=False),
        scratch_types=[
            pltpu.VMEM((embed, 2 * CHUNK), jnp.float32),
            pltpu.VMEM((embed, 2 * CHUNK), jnp.float32),
            pltpu.VMEM((CHUNK * CHUNK,), jnp.float32),
            pltpu.VMEM((CHUNK * CHUNK,), jnp.float32),
            pltpu.VMEM((embed, tail if tail else 16), jnp.float32),
            pltpu.SemaphoreType.DMA,
            pltpu.SemaphoreType.DMA,
            pltpu.SemaphoreType.DMA,
            pltpu.SemaphoreType.DMA,
        ],
    )


def _make_gather(vocab, embed, nchunks):
    mesh = plsc.VectorSubcoreMesh(
        core_axis_name="c", subcore_axis_name="s",
        num_cores=NC, num_subcores=NS)

    def body(table_hbm, idx_hbm, out_hbm, idx_v, bufs, gsems, osems):
        wid = lax.axis_index("s") * NC + lax.axis_index("c")
        pltpu.sync_copy(idx_hbm.at[wid], idx_v)
        row_base = wid * nchunks * CHUNK

        def gather_copy(j, slot):
            return pltpu.make_async_copy(
                table_hbm.at[idx_v.at[j]], bufs.at[slot], gsems.at[slot])

        def out_copy(j, slot):
            return pltpu.make_async_copy(
                bufs.at[slot],
                out_hbm.at[pl.ds(row_base + j * CHUNK, CHUNK)],
                osems.at[slot])

        for b in range(LEAD):
            gather_copy(b, b).start()

        @pl.loop(0, nchunks, step=NBUF)
        def _(j0):
            for b in range(NBUF):
                j = j0 + b
                slot_ahead = (b + LEAD) % NBUF

                @pl.when(j >= LEAD)
                def _():
                    out_copy(j - LEAD, slot_ahead).wait()

                @pl.when(j + LEAD < nchunks)
                def _():
                    gather_copy(j + LEAD, slot_ahead).start()

                gather_copy(j, b).wait()
                out_copy(j, b).start()

        for j in range(nchunks - LEAD, nchunks):
            out_copy(j, j % NBUF).wait()

    return pl.kernel(
        body,
        out_type=jax.ShapeDtypeStruct((NW * nchunks * CHUNK, embed),
                                      jnp.float32),
        mesh=mesh,
        compiler_params=pltpu.CompilerParams(use_tc_tiling_on_sc=True),
        scratch_types=[
            pltpu.VMEM((nchunks, CHUNK), jnp.int32),
            pltpu.VMEM((NBUF, CHUNK, embed), jnp.float32),
            pltpu.SemaphoreType.DMA((NBUF,)),
            pltpu.SemaphoreType.DMA((NBUF,)),
        ],
    )


def kernel(inputs, table):
    b, s = inputs.shape
    vocab, embed = table.shape
    total = b * s
    nchunks = total // (NW * CHUNK)
    idx = inputs.reshape(NW, nchunks, CHUNK).astype(jnp.int32)
    table_p = _make_transpose(vocab, embed)(table.T).reshape(vocab, 128)
    out = _make_gather(vocab, 128, nchunks)(table_p, idx)
    return out[:, :embed].reshape(b, s, embed)


# speedup vs baseline: 3.0867x; 1.4757x over previous
import jax
import jax.numpy as jnp
from jax import lax
from jax.experimental import pallas as pl
from jax.experimental.pallas import tpu as pltpu
from jax.experimental.pallas import tpu_sc as plsc

NC, NS = 2, 16
NW = NC * NS
CHUNK = 128
NBUF = 4
LEAD = 2


def _make_transpose(vocab, embed):
    nfull = vocab // CHUNK
    tail = vocab - nfull * CHUNK
    nb, nx = divmod(nfull, NW)
    if nb % 2:
        nb -= 1
        nx += NW
    mesh = plsc.VectorSubcoreMesh(
        core_axis_name="c", subcore_axis_name="s",
        num_cores=NC, num_subcores=NS)

    def body(tt_hbm, tp_hbm, tin0, tin1, tout0, tout1, tin_tail,
             is0, is1, os0, os1):
        w = lax.axis_index("s") * NC + lax.axis_index("c")
        lane = lax.iota(jnp.int32, 16)
        lane128 = lane * CHUNK
        rot = [(lane + d) % 16 for d in range(16)]

        vb0 = w * nb

        def in_copy(s, tinb, sem):
            return pltpu.make_async_copy(
                tt_hbm.at[:, pl.ds((vb0 + 2 * s) * CHUNK, 2 * CHUNK)],
                tinb, sem)

        def out_copy(vb, toutb, sem):
            return pltpu.make_async_copy(
                toutb, tp_hbm.at[pl.ds(vb * CHUNK * CHUNK, CHUNK * CHUNK)],
                sem)

        def transpose_block(tinb, toutb, ngroups, loff=0, compact=False):
            if compact:
                shift = ngroups.bit_length() - 1

                @plsc.parallel_loop(0, ngroups * embed, unroll=8)
                def _(t):
                    e = t >> shift
                    lg = t & (ngroups - 1)
                    x = tinb[e, pl.ds(loff + lg * 16, 16)]
                    plsc.store_scatter(toutb, [lane128 + (lg << 11) + e], x)
            else:
                nsub = embed // 16

                @plsc.parallel_loop(0, ngroups * nsub, unroll=2)
                def _(st):
                    e0 = (st % nsub) * 16
                    l0 = (st // nsub) * 16
                    for d in range(16):
                        x = plsc.load_gather(
                            tinb, [e0 + lane, loff + l0 + rot[d]])
                        plsc.store_scatter(
                            toutb, [(l0 + rot[d]) * CHUNK + e0 + lane], x)

        zeros16 = jnp.zeros((16,), jnp.float32)

        @pl.loop(0, CHUNK)
        def _(l):
            for toutb in (tout0, tout1):
                for k in range((CHUNK - embed) // 16):
                    toutb[pl.ds(l * CHUNK + embed + 16 * k, 16)] = zeros16

        nsb = nb // 2
        in_copy(0, tin0, is0).start()
        in_copy(1, tin1, is1).start()

        @pl.loop(0, nsb, step=2)
        def _(s0):
            for b, tinb, isem in ((0, tin0, is0), (1, tin1, is1)):
                s = s0 + b
                in_copy(s, tinb, isem).wait()
                for j, toutb, osem in ((0, tout0, os0), (1, tout1, os1)):
                    @pl.when(s >= 1)
                    def _(s=s, j=j, toutb=toutb, osem=osem):
                        out_copy(vb0 + 2 * (s - 1) + j, toutb, osem).wait()

                    transpose_block(tinb, toutb, 8, loff=j * CHUNK)
                    out_copy(vb0 + 2 * s + j, toutb, osem).start()

                @pl.when(s + 2 < nsb)
                def _(s=s, tinb=tinb, isem=isem):
                    in_copy(s + 2, tinb, isem).start()

        out_copy(vb0 + nb - 2, tout0, os0).wait()
        out_copy(vb0 + nb - 1, tout1, os1).wait()

        @pl.when(w < nx)
        def _():
            vbx = nfull - nx + w
            pltpu.sync_copy(tt_hbm.at[:, pl.ds(vbx * CHUNK, CHUNK)],
                            tin0.at[:, pl.ds(0, CHUNK)])
            transpose_block(tin0, tout0, 8, compact=True)
            pltpu.sync_copy(
                tout0, tp_hbm.at[pl.ds(vbx * CHUNK * CHUNK, CHUNK * CHUNK)])

        if tail:
            @pl.when(w == nx)
            def _():
                pltpu.sync_copy(tt_hbm.at[:, pl.ds(nfull * CHUNK, tail)],
                                tin_tail)
                transpose_block(tin_tail, tout1, tail // 16, compact=True)
                pltpu.sync_copy(
                    tout1.at[pl.ds(0, tail * CHUNK)],
                    tp_hbm.at[pl.ds(nfull * CHUNK * CHUNK, tail * CHUNK)])

    return pl.kernel(
        body,
        out_type=jax.ShapeDtypeStruct((vocab * CHUNK,), jnp.float32),
        mesh=mesh,
        compiler_params=pltpu.CompilerParams(use_tc_tiling_on_sc=True,
                                             needs_layout_passes=False),
        scratch_types=[
            pltpu.VMEM((embed, 2 * CHUNK), jnp.float32),
            pltpu.VMEM((embed, 2 * CHUNK), jnp.float32),
            pltpu.VMEM((CHUNK * CHUNK,), jnp.float32),
            pltpu.VMEM((CHUNK * CHUNK,), jnp.float32),
            pltpu.VMEM((embed, tail if tail else 16), jnp.float32),
            pltpu.SemaphoreType.DMA,
            pltpu.SemaphoreType.DMA,
            pltpu.SemaphoreType.DMA,
            pltpu.SemaphoreType.DMA,
        ],
    )


def _make_gather(vocab, embed, nchunks):
    mesh = plsc.VectorSubcoreMesh(
        core_axis_name="c", subcore_axis_name="s",
        num_cores=NC, num_subcores=NS)

    def body(table_hbm, idx_hbm, x_hbm, idx_v, bufs, xo0, xo1,
             gsems, os0, os1):
        wid = lax.axis_index("s") * NC + lax.axis_index("c")
        lane = lax.iota(jnp.int32, 16)
        rot = [(lane + d) % 16 for d in range(16)]
        pltpu.sync_copy(idx_hbm.at[:, pl.ds(wid * CHUNK, CHUNK)], idx_v)

        def gather_copy(j, slot):
            return pltpu.make_async_copy(
                table_hbm.at[idx_v.at[j]], bufs.at[slot], gsems.at[slot])

        def x_copy(j, xob, sem):
            return pltpu.make_async_copy(
                xob, x_hbm.at[j, :, pl.ds(wid * CHUNK, CHUNK)], sem)

        def transpose_panel(slot, xob):
            @plsc.parallel_loop(0, (embed // 16) * 8, unroll=2)
            def _(st):
                e0 = (st % (embed // 16)) * 16
                l0 = (st // (embed // 16)) * 16
                for d in range(16):
                    x = plsc.load_gather(
                        bufs.at[slot], [l0 + rot[d], e0 + lane])
                    plsc.store_scatter(xob, [e0 + lane, l0 + rot[d]], x)

        gather_copy(0, 0).start()
        gather_copy(1, 1).start()

        @pl.loop(0, nchunks, step=NBUF)
        def _(j0):
            for b in range(NBUF):
                j = j0 + b
                xob, osem = (xo0, os0) if b % 2 == 0 else (xo1, os1)

                @pl.when(j + LEAD < nchunks)
                def _(j=j, b=b):
                    gather_copy(j + LEAD, (b + LEAD) % NBUF).start()

                gather_copy(j, b).wait()

                @pl.when(j >= 2)
                def _(j=j, xob=xob, osem=osem):
                    x_copy(j - 2, xob, osem).wait()

                transpose_panel(b, xob)
                x_copy(j, xob, osem).start()

        x_copy(nchunks - 2, xo0, os0).wait()
        x_copy(nchunks - 1, xo1, os1).wait()

    return pl.kernel(
        body,
        out_type=jax.ShapeDtypeStruct((nchunks, embed, NW * CHUNK),
                                      jnp.float32),
        mesh=mesh,
        compiler_params=pltpu.CompilerParams(use_tc_tiling_on_sc=True,
                                             needs_layout_passes=False),
        scratch_types=[
            pltpu.VMEM((nchunks, CHUNK), jnp.int32),
            pltpu.VMEM((NBUF, CHUNK, 128), jnp.float32),
            pltpu.VMEM((embed, CHUNK), jnp.float32),
            pltpu.VMEM((embed, CHUNK), jnp.float32),
            pltpu.SemaphoreType.DMA((NBUF,)),
            pltpu.SemaphoreType.DMA,
            pltpu.SemaphoreType.DMA,
        ],
    )


def kernel(inputs, table):
    b, s = inputs.shape
    vocab, embed = table.shape
    table_p = _make_transpose(vocab, embed)(table.T).reshape(vocab, 128)
    idx_t = inputs.T.astype(jnp.int32)
    x = _make_gather(vocab, embed, s)(table_p, idx_t)
    return jnp.transpose(x, (2, 0, 1))
